# baseline (speedup 1.0000x reference)
import functools
import math

import jax
import jax.numpy as jnp
import numpy as np
from jax.experimental import pallas as pl
from jax.experimental.pallas import tpu as pltpu

_NOISE_ROWS = 64


def _round_up(a, b):
    return ((a + b - 1) // b) * b


def _reduce_kernel(x_ref, o_ref):
    xf = x_ref[...].astype(jnp.float32)
    r = jnp.sum(xf * xf, axis=1, keepdims=True)
    s = jnp.sum(r, axis=0, keepdims=True)
    o_ref[...] = jnp.broadcast_to(s.reshape(1, 1, 1), o_ref.shape)


def _apply_kernel(sums_ref, x_ref, o_ref, *, inv_n, std, chunks, seed):
    i = pl.program_id(0)
    total = jnp.sum(sums_ref[:, 0, :1], axis=0, keepdims=True)
    inv = jax.lax.rsqrt(total * (2.0 * inv_n))
    for j in range(chunks):
        t = i * chunks + j
        pltpu.prng_seed(seed + t)
        noise = pltpu.stateful_normal((_NOISE_ROWS, x_ref.shape[1]),
                                      jnp.float32)
        sl = pl.ds(j * _NOISE_ROWS, _NOISE_ROWS)
        xf = x_ref[sl, :].astype(jnp.float32)
        o_ref[sl, :] = (xf * inv + noise * std).astype(o_ref.dtype)


def _add_noise(x, s, seed=0):
    std = float(math.sqrt(s))
    orig_shape = x.shape
    orig_dtype = x.dtype
    n = int(np.prod(orig_shape))
    inv_n = 1.0 / n

    W = 128
    for cand in (2048, 1024, 512, 256, 128):
        if n % cand == 0 and (n // cand) % 8 == 0:
            W = cand
            break
    rows = pl.cdiv(n, W)
    tile_rows = min(max(8, (512 * 1024) // (4 * W)), _round_up(rows, 8))
    rows_padded = _round_up(rows, tile_rows)
    total = rows_padded * W

    flat = jnp.ravel(x)
    if total != n:
        flat = jnp.pad(flat, (0, total - n))
    mat = flat.reshape(rows_padded, W)

    red_rows = tile_rows
    for cand in (1024, 512, 256, 128, 64):
        if rows_padded % cand == 0 and cand >= tile_rows:
            red_rows = cand
            break
    n_red = rows_padded // red_rows
    partials = pl.pallas_call(
        _reduce_kernel,
        out_shape=jax.ShapeDtypeStruct((n_red, 8, 128), jnp.float32),
        grid=(n_red,),
        in_specs=[pl.BlockSpec((red_rows, W), lambda b: (b, 0))],
        out_specs=pl.BlockSpec((1, 8, 128), lambda b: (b, 0, 0)),
        compiler_params=pltpu.CompilerParams(
            dimension_semantics=("parallel",),
            vmem_limit_bytes=64 * 1024 * 1024,
        ),
    )(mat)

    chunks = 8
    while (rows_padded // _NOISE_ROWS) % chunks:
        chunks //= 2
    blk_rows = _NOISE_ROWS * chunks
    n_blk = rows_padded // blk_rows
    out = pl.pallas_call(
        functools.partial(_apply_kernel, inv_n=inv_n, std=std,
                          chunks=chunks, seed=int(seed)),
        out_shape=jax.ShapeDtypeStruct((rows_padded, W), orig_dtype),
        grid=(n_blk,),
        in_specs=[
            pl.BlockSpec((n_red, 8, 128), lambda b: (0, 0, 0)),
            pl.BlockSpec((blk_rows, W), lambda b: (b, 0)),
        ],
        out_specs=pl.BlockSpec((blk_rows, W), lambda b: (b, 0)),
        compiler_params=pltpu.CompilerParams(
            dimension_semantics=("parallel",),
            vmem_limit_bytes=64 * 1024 * 1024,
        ),
    )(partials, mat)

    if total == n:
        return out.reshape(orig_shape)
    return out.reshape(-1)[:n].reshape(orig_shape)


def kernel(x):
    return _add_noise(x, 0.1, seed=0)
